# Initial kernel scaffold; baseline (speedup 1.0000x reference)
import jax
import jax.numpy as jnp
from jax.experimental import pallas as pl


def kernel(sm_mask, labels):
    raise NotImplementedError("write your pallas kernel here")


# speedup vs baseline: 35.3335x; 35.3335x over previous
import functools

import jax
import jax.numpy as jnp
import numpy as np
from jax import lax
from jax.experimental import pallas as pl
from jax.experimental.pallas import tpu as pltpu

TOTAL_PIX = 41 * 41
NSORT = 2048
ROWS_PER_BLOCK = 168
N_CLASSES = 21

_wfg = np.array([0.996 ** i for i in range(TOTAL_PIX)], dtype=np.float64)
_wfg = (_wfg / _wfg.sum()).astype(np.float32)
_wbg = np.array([0.999 ** i for i in range(TOTAL_PIX)], dtype=np.float64)
_wbg = (_wbg / _wbg.sum()).astype(np.float32)
W_FG_PAD = jnp.asarray(np.pad(_wfg, (0, NSORT - TOTAL_PIX))).reshape(1, NSORT)
W_BG_PAD = jnp.asarray(np.pad(_wbg, (0, NSORT - TOTAL_PIX))).reshape(1, NSORT)


def _bitonic_sort_desc(x):
    n = x.shape[-1]
    lane = lax.broadcasted_iota(jnp.int32, (1, n), 1)
    log_n = n.bit_length() - 1
    for k in range(log_n):
        for j in range(k, -1, -1):
            d = 1 << j
            islow = (lane & d) == 0
            desc_blk = (lane & (1 << (k + 1))) == 0
            keep_max = islow == desc_blk
            partner = jnp.where(islow, jnp.roll(x, -d, axis=-1),
                                jnp.roll(x, d, axis=-1))
            x = jnp.where(keep_max, jnp.maximum(x, partner),
                          jnp.minimum(x, partner))
    return x


def _block_body(x_ref, lab_ref, wfg_ref, wbg_ref, out_ref):
    x = x_ref[...]
    xs = _bitonic_sort_desc(x)
    fg = jnp.sum(xs * wfg_ref[...], axis=-1)
    bg = jnp.sum(xs * wbg_ref[...], axis=-1)
    mx = jnp.max(x, axis=-1)

    nb = ROWS_PER_BLOCK // N_CLASSES
    lab = lab_ref[...].reshape(1, ROWS_PER_BLOCK)
    fg2 = fg.reshape(1, ROWS_PER_BLOCK)
    bg2 = bg.reshape(1, ROWS_PER_BLOCK)
    mx2 = mx.reshape(1, ROWS_PER_BLOCK)

    col = lax.broadcasted_iota(jnp.int32, (1, ROWS_PER_BLOCK), 1) % N_CLASSES
    present = lab != 0
    labf = lab.astype(jnp.float32)

    bg_term = jnp.sum(jnp.where(present & (col == 0), -jnp.log(bg2), 0.0))

    bidx = lax.broadcasted_iota(jnp.int32, (nb, ROWS_PER_BLOCK), 0)
    ridx = (lax.broadcasted_iota(jnp.int32, (nb, ROWS_PER_BLOCK), 1)
            // N_CLASSES)
    seg = bidx == ridx

    def seg_sum(v):
        return jnp.sum(jnp.where(seg, v, 0.0), axis=1)

    fg_sum = seg_sum(jnp.where(present & (col != 0), -jnp.log(fg2), 0.0))
    n_fg = seg_sum(jnp.where(col != 0, labf, 0.0))
    nx_sum = seg_sum(jnp.where(~present, -jnp.log(mx2), 0.0))
    n_nx = seg_sum(jnp.where(present, 0.0, 1.0))

    total = bg_term + jnp.sum(fg_sum / n_fg) + jnp.sum(nx_sum / n_nx)

    @pl.when(pl.program_id(0) == 0)
    def _():
        out_ref[0, 0] = 0.0

    out_ref[0, 0] += total


def _run(x_pad, lab3, wfg, wbg, batch_size, interpret=False):
    n_rows = x_pad.shape[0]
    grid = n_rows // ROWS_PER_BLOCK
    out = pl.pallas_call(
        _block_body,
        grid=(grid,),
        in_specs=[
            pl.BlockSpec((ROWS_PER_BLOCK, NSORT), lambda i: (i, 0)),
            pl.BlockSpec((1, 1, ROWS_PER_BLOCK), lambda i: (i, 0, 0)),
            pl.BlockSpec((1, NSORT), lambda i: (0, 0)),
            pl.BlockSpec((1, NSORT), lambda i: (0, 0)),
        ],
        out_specs=pl.BlockSpec((1, 1), lambda i: (0, 0),
                               memory_space=pltpu.SMEM),
        out_shape=jax.ShapeDtypeStruct((1, 1), jnp.float32),
        interpret=interpret,
    )(x_pad, lab3, wfg, wbg)
    return out[0, 0] / batch_size


@jax.jit
def kernel(sm_mask, labels):
    batch_size, n_classes = labels.shape
    n_rows = batch_size * n_classes
    x = sm_mask.reshape(n_rows, TOTAL_PIX)
    x_pad = jnp.pad(x, ((0, 0), (0, NSORT - TOTAL_PIX)),
                    constant_values=-1.0)
    grid = n_rows // ROWS_PER_BLOCK
    lab3 = labels.reshape(grid, 1, ROWS_PER_BLOCK)
    return _run(x_pad, lab3, W_FG_PAD, W_BG_PAD, batch_size)
